# baseline (speedup 1.0000x reference)
import functools

import jax
import jax.numpy as jnp
from jax import lax
from jax.experimental import pallas as pl
from jax.experimental.pallas import tpu as pltpu
from jax.experimental.pallas import tpu_sc as plsc

D_IN = 64
D_H = 128
B_TOK = 16384 * 20
NW = 32
BPW = B_TOK // NW
CH = 512
NCHUNK = BPW // CH


def _sc_gather(table, idx3):
  mesh = plsc.VectorSubcoreMesh(core_axis_name="c", subcore_axis_name="s")

  @functools.partial(
      pl.kernel,
      mesh=mesh,
      compiler_params=pltpu.CompilerParams(use_tc_tiling_on_sc=False),
      out_type=jax.ShapeDtypeStruct((B_TOK, D_IN), jnp.float32),
      scratch_types=[
          pltpu.VMEM((NCHUNK, CH), jnp.int32),
          pltpu.VMEM((CH, D_IN), jnp.float32),
          pltpu.VMEM((CH, D_IN), jnp.float32),
          pltpu.SemaphoreType.DMA,
          pltpu.SemaphoreType.DMA,
      ],
  )
  def k(table_hbm, idx_hbm, out_hbm, idx_v, buf0, buf1, sem0, sem1):
    wid = lax.axis_index("s") * 2 + lax.axis_index("c")
    base = wid * BPW
    pltpu.sync_copy(idx_hbm.at[wid], idx_v)
    bufs = (buf0, buf1)
    sems = (sem0, sem1)
    cps = [None, None]
    cps[0] = pltpu.async_copy(table_hbm.at[idx_v.at[0]], buf0, sem0)
    for c in range(NCHUNK):
      nxt = c + 1
      if nxt < NCHUNK:
        cps[nxt % 2] = pltpu.async_copy(
            table_hbm.at[idx_v.at[nxt]], bufs[nxt % 2], sems[nxt % 2])
      cps[c % 2].wait()
      pltpu.sync_copy(bufs[c % 2], out_hbm.at[pl.ds(base + c * CH, CH)])

  return k(table, idx3)


def _tc_mlp(emb, W1, b1, W2, b2):
  BLK = 2048

  def body(e_ref, w1_ref, b1_ref, w2_ref, b2_ref, o_ref):
    e = e_ref[...]
    h = jnp.dot(e, w1_ref[...], preferred_element_type=jnp.float32) + b1_ref[...]
    h = h * jax.nn.sigmoid(h)
    o_ref[...] = (
        jnp.dot(h, w2_ref[...], preferred_element_type=jnp.float32) + b2_ref[...]
    )

  return pl.pallas_call(
      body,
      grid=(B_TOK // BLK,),
      in_specs=[
          pl.BlockSpec((BLK, D_IN), lambda i: (i, 0)),
          pl.BlockSpec((D_IN, D_H), lambda i: (0, 0)),
          pl.BlockSpec((1, D_H), lambda i: (0, 0)),
          pl.BlockSpec((D_H, D_H), lambda i: (0, 0)),
          pl.BlockSpec((1, D_H), lambda i: (0, 0)),
      ],
      out_specs=pl.BlockSpec((BLK, D_H), lambda i: (i, 0)),
      out_shape=jax.ShapeDtypeStruct((B_TOK, D_H), jnp.float32),
  )(emb, W1, b1, W2, b2)


def kernel(t, table, W1, b1, W2, b2):
  Bt, L = t.shape
  idx3 = t.reshape(NW, NCHUNK, CH)
  emb = _sc_gather(table, idx3)
  out = _tc_mlp(emb, W1, b1.reshape(1, D_H), W2, b2.reshape(1, D_H))
  return out.reshape(Bt, L, D_H)


# speedup vs baseline: 1.3571x; 1.3571x over previous
import functools

import jax
import jax.numpy as jnp
from jax import lax
from jax.experimental import pallas as pl
from jax.experimental.pallas import tpu as pltpu
from jax.experimental.pallas import tpu_sc as plsc

D_IN = 64
D_H = 128
B_TOK = 16384 * 20
NW = 32
BPW = B_TOK // NW
CH = 512
NCHUNK = BPW // CH


def _sc_gather(table, idx3):
  mesh = plsc.VectorSubcoreMesh(core_axis_name="c", subcore_axis_name="s")

  @functools.partial(
      pl.kernel,
      mesh=mesh,
      compiler_params=pltpu.CompilerParams(use_tc_tiling_on_sc=False),
      out_type=jax.ShapeDtypeStruct((B_TOK, D_IN), jnp.float32),
      scratch_types=[
          pltpu.VMEM((NCHUNK, CH), jnp.int32),
          pltpu.VMEM((CH, D_IN), jnp.float32),
          pltpu.VMEM((CH, D_IN), jnp.float32),
          pltpu.SemaphoreType.DMA,
          pltpu.SemaphoreType.DMA,
      ],
  )
  def k(table_hbm, idx_hbm, out_hbm, idx_v, buf0, buf1, sem0, sem1):
    wid = lax.axis_index("s") * 2 + lax.axis_index("c")
    base = wid * BPW
    pltpu.sync_copy(idx_hbm.at[wid], idx_v)
    bufs = (buf0, buf1)
    sems = (sem0, sem1)
    cps = [None, None]
    cps[0] = pltpu.async_copy(table_hbm.at[idx_v.at[0]], buf0, sem0)
    for c in range(NCHUNK):
      nxt = c + 1
      if nxt < NCHUNK:
        cps[nxt % 2] = pltpu.async_copy(
            table_hbm.at[idx_v.at[nxt]], bufs[nxt % 2], sems[nxt % 2])
      cps[c % 2].wait()
      pltpu.sync_copy(bufs[c % 2], out_hbm.at[pl.ds(base + c * CH, CH)])

  return k(table, idx3)


def _tc_mlp(emb2, W1b, b1b, W2b, b2b):
  BLK = 2048
  n_rows = emb2.shape[0]

  def body(e_ref, w1_ref, b1_ref, w2_ref, b2_ref, o_ref):
    e = e_ref[...]
    h = jnp.dot(e, w1_ref[...], preferred_element_type=jnp.float32) + b1_ref[...]
    h = h * jax.nn.sigmoid(h)
    o_ref[...] = (
        jnp.dot(h, w2_ref[...], preferred_element_type=jnp.float32) + b2_ref[...]
    )

  return pl.pallas_call(
      body,
      grid=(n_rows // BLK,),
      in_specs=[
          pl.BlockSpec((BLK, 2 * D_IN), lambda i: (i, 0)),
          pl.BlockSpec((2 * D_IN, 2 * D_H), lambda i: (0, 0)),
          pl.BlockSpec((1, 2 * D_H), lambda i: (0, 0)),
          pl.BlockSpec((2 * D_H, 2 * D_H), lambda i: (0, 0)),
          pl.BlockSpec((1, 2 * D_H), lambda i: (0, 0)),
      ],
      out_specs=pl.BlockSpec((BLK, 2 * D_H), lambda i: (i, 0)),
      out_shape=jax.ShapeDtypeStruct((n_rows, 2 * D_H), jnp.float32),
  )(emb2, W1b, b1b, W2b, b2b)


def kernel(t, table, W1, b1, W2, b2):
  Bt, L = t.shape
  idx3 = t.T.reshape(NW, NCHUNK, CH)
  emb = _sc_gather(table, idx3)
  emb2 = emb.reshape(B_TOK // 2, 2 * D_IN)
  Z = jnp.zeros_like(W1)
  W1b = jnp.block([[W1, Z], [Z, W1]])
  Zh = jnp.zeros_like(W2)
  W2b = jnp.block([[W2, Zh], [Zh, W2]])
  b1b = jnp.concatenate([b1, b1]).reshape(1, 2 * D_H)
  b2b = jnp.concatenate([b2, b2]).reshape(1, 2 * D_H)
  out2 = _tc_mlp(emb2, W1b, b1b, W2b, b2b)
  return out2.reshape(L, Bt, D_H).transpose(1, 0, 2)
